# Initial kernel scaffold; baseline (speedup 1.0000x reference)
import jax
import jax.numpy as jnp
from jax.experimental import pallas as pl


def kernel(prediction, target):
    raise NotImplementedError("write your pallas kernel here")


# speedup vs baseline: 5.2099x; 5.2099x over previous
import functools

import jax
import jax.numpy as jnp
from jax import lax
from jax.experimental import pallas as pl
from jax.experimental.pallas import tpu as pltpu
from jax.experimental.pallas import tpu_sc as plsc

C = 19
H = 1024
W = 2048
N = H * W
K = int(0.3 * N)
EPS = 1e-9

NC = 2
NS = 16
NW = NC * NS
NPW = N // NW
CH = NPW // 2
GB = 128
GROUP = 8


def _sc_gather(pred_flat, tgt_flat):
    mesh = plsc.VectorSubcoreMesh(core_axis_name="c", subcore_axis_name="s")

    @functools.partial(
        pl.kernel,
        out_type=jax.ShapeDtypeStruct((N,), jnp.float32),
        mesh=mesh,
        scratch_types=[
            pltpu.VMEM((CH,), jnp.int32),
            pltpu.VMEM((NPW,), jnp.float32),
            pltpu.SemaphoreType.DMA,
        ],
    )
    def k(pred_hbm, tgt_hbm, out_hbm, idx_v, vals_v, sem):
        wid = lax.axis_index("s") * NC + lax.axis_index("c")
        base = wid * NPW
        lane = lax.iota(jnp.int32, 16)
        for h in range(2):
            hbase = base + h * CH
            pltpu.sync_copy(tgt_hbm.at[pl.ds(hbase, CH)], idx_v)

            def idx_body(i, carry):
                sl = pl.ds(i * 16, 16)
                t = idx_v[sl]
                idx_v[sl] = t * N + (hbase + i * 16) + lane
                return carry

            lax.fori_loop(0, CH // 16, idx_body, 0)

            def gather_body(g, carry):
                descs = []
                for j in range(GROUP):
                    off = (g * GROUP + j) * GB
                    descs.append(pltpu.async_copy(
                        pred_hbm.at[idx_v.at[pl.ds(off, GB)]],
                        vals_v.at[pl.ds(h * CH + off, GB)],
                        sem,
                    ))
                for d in descs:
                    d.wait()
                return carry

            lax.fori_loop(0, CH // (GB * GROUP), gather_body, 0)

        pltpu.sync_copy(vals_v, out_hbm.at[pl.ds(base, NPW)])

    return k(pred_flat, tgt_flat)


NB = 4
BR = H // NB


def _tc_body(vals_ref, out_ref, prefix_ref, kr_ref, cnt_ref, acc_ref):
    p = pl.program_id(0)
    b = pl.program_id(1)

    @pl.when(jnp.logical_and(p == 0, b == 0))
    def _():
        prefix_ref[0] = 0
        kr_ref[0] = K
        cnt_ref[0] = 0
        acc_ref[0] = 0.0

    v = vals_ref[...]
    bits = lax.bitcast_convert_type(v, jnp.int32)

    @pl.when(p < 31)
    def _():
        bit = 30 - p
        pfx = prefix_ref[0]
        hi_match = (bits >> (bit + 1)) == (pfx >> (bit + 1))
        zero_bit = ((bits >> bit) & 1) == 0
        c_step = jnp.sum(jnp.logical_and(hi_match, zero_bit).astype(jnp.int32))
        cnt_ref[0] = cnt_ref[0] + c_step

        @pl.when(b == NB - 1)
        def _():
            c = cnt_ref[0]
            take1 = kr_ref[0] > c
            prefix_ref[0] = jnp.where(take1, pfx | (1 << bit), pfx)
            kr_ref[0] = jnp.where(take1, kr_ref[0] - c, kr_ref[0])
            cnt_ref[0] = 0

    @pl.when(p == 31)
    def _():
        pfx = prefix_ref[0]
        s = jnp.sum(jnp.where(bits < pfx, -jnp.log(v + EPS), 0.0))
        acc_ref[0] = acc_ref[0] + s

        @pl.when(b == NB - 1)
        def _():
            vt = lax.bitcast_convert_type(
                jnp.full((1, 1), pfx, jnp.int32), jnp.float32)
            tie_loss = -jnp.log(vt + EPS)[0, 0]
            ties = kr_ref[0].astype(jnp.float32)
            out_ref[0, 0] = (acc_ref[0] + ties * tie_loss) / K


def _tc_select_sum(vals2d):
    return pl.pallas_call(
        _tc_body,
        grid=(32, NB),
        in_specs=[pl.BlockSpec((BR, W), lambda p, b: (b, 0))],
        out_specs=pl.BlockSpec(memory_space=pltpu.SMEM),
        out_shape=jax.ShapeDtypeStruct((1, 1), jnp.float32),
        scratch_shapes=[
            pltpu.SMEM((1,), jnp.int32),
            pltpu.SMEM((1,), jnp.int32),
            pltpu.SMEM((1,), jnp.int32),
            pltpu.SMEM((1,), jnp.float32),
        ],
    )(vals2d)


def kernel(prediction, target):
    pred_flat = prediction.reshape(C * N)
    tgt_flat = target.reshape(N)
    vals = _sc_gather(pred_flat, tgt_flat)
    out = _tc_select_sum(vals.reshape(H, W))
    return out[0, 0]
